# Initial kernel scaffold; baseline (speedup 1.0000x reference)
import jax
import jax.numpy as jnp
from jax.experimental import pallas as pl


def kernel(i, table):
    raise NotImplementedError("write your pallas kernel here")


# speedup vs baseline: 1.4960x; 1.4960x over previous
import functools

import jax
import jax.numpy as jnp
from jax import lax
from jax.experimental import pallas as pl
from jax.experimental.pallas import tpu as pltpu
from jax.experimental.pallas import tpu_sc as plsc

_DIM = 32
_NC, _NS = 2, 16
_NW = _NC * _NS

_mesh = plsc.VectorSubcoreMesh(
    core_axis_name="c", subcore_axis_name="s",
    num_cores=_NC, num_subcores=_NS)


@functools.partial(jax.jit, static_argnums=(2,))
def _gather_rows(idx_flat, table, n_total):
  b_per_w = n_total // _NW
  chunk = 3200
  n_chunks = b_per_w // chunk

  @functools.partial(
      pl.kernel,
      out_type=jax.ShapeDtypeStruct((n_total, _DIM), jnp.float32),
      mesh=_mesh,
      scratch_types=[
          pltpu.VMEM((chunk,), jnp.int32),
          pltpu.VMEM((chunk, _DIM), jnp.float32),
          pltpu.SemaphoreType.DMA,
      ],
      compiler_params=pltpu.CompilerParams(use_tc_tiling_on_sc=False),
  )
  def gather_kernel(idx_hbm, table_hbm, out_hbm, idx_v, rows_v, sem):
    wid = lax.axis_index("s") * _NC + lax.axis_index("c")
    base = wid * b_per_w

    def body(g, carry):
      off = base + g * chunk
      pltpu.sync_copy(idx_hbm.at[pl.ds(off, chunk)], idx_v)
      pltpu.async_copy(table_hbm.at[idx_v], rows_v, sem).wait()
      pltpu.sync_copy(rows_v, out_hbm.at[pl.ds(off, chunk)])
      return carry

    lax.fori_loop(0, n_chunks, body, 0)

  return gather_kernel(idx_flat, table)


def kernel(i, table):
  flat = i.reshape(-1)
  out = _gather_rows(flat, table, flat.shape[0])
  return out.reshape(i.shape + (table.shape[-1],))
